# baseline (speedup 1.0000x reference)
import jax
import jax.numpy as jnp
from jax import lax
from jax.experimental import pallas as pl
from jax.experimental.pallas import tpu as pltpu
from jax.experimental.pallas import tpu_sc as plsc

NUM_WORKERS = 32
K = 4


def _gather_body(x_hbm, w_hbm, out_hbm, idx_v, buf_a, buf_b, gsem_a, gsem_b,
                 wsem_a, wsem_b):
  rows_per_w = x_hbm.shape[0] // NUM_WORKERS
  ng = rows_per_w // K
  c = lax.axis_index("c")
  s = lax.axis_index("s")
  wid = s * 2 + c
  base = wid * rows_per_w
  pltpu.sync_copy(x_hbm.at[pl.ds(base, rows_per_w)], idx_v)

  def start_gathers(g, buf, sem):
    for b in range(K):
      pltpu.async_copy(w_hbm.at[idx_v.at[g * K + b]], buf.at[b], sem)

  def wait_gathers(g, buf, sem):
    for b in range(K):
      pltpu.make_async_copy(w_hbm.at[idx_v.at[g * K + b]], buf.at[b],
                            sem).wait()

  def start_writes(g, buf, sem):
    for b in range(K):
      pltpu.async_copy(buf.at[b], out_hbm.at[base + g * K + b], sem)

  def wait_writes(g, buf, sem):
    for b in range(K):
      pltpu.make_async_copy(buf.at[b], out_hbm.at[base + g * K + b],
                            sem).wait()

  start_gathers(0, buf_a, gsem_a)
  wait_gathers(0, buf_a, gsem_a)
  start_writes(0, buf_a, wsem_a)
  start_gathers(1, buf_b, gsem_b)
  wait_gathers(1, buf_b, gsem_b)
  start_writes(1, buf_b, wsem_b)
  wait_writes(0, buf_a, wsem_a)
  start_gathers(2, buf_a, gsem_a)

  def pair(q, carry):
    g = 2 * q + 2
    wait_gathers(g, buf_a, gsem_a)
    start_writes(g, buf_a, wsem_a)
    wait_writes(g - 1, buf_b, wsem_b)
    start_gathers(g + 1, buf_b, gsem_b)
    wait_gathers(g + 1, buf_b, gsem_b)
    start_writes(g + 1, buf_b, wsem_b)
    wait_writes(g, buf_a, wsem_a)
    start_gathers(g + 2, buf_a, gsem_a)
    return carry

  lax.fori_loop(0, (ng - 4) // 2, pair, 0)

  wait_gathers(ng - 2, buf_a, gsem_a)
  start_writes(ng - 2, buf_a, wsem_a)
  wait_writes(ng - 3, buf_b, wsem_b)
  start_gathers(ng - 1, buf_b, gsem_b)
  wait_gathers(ng - 1, buf_b, gsem_b)
  start_writes(ng - 1, buf_b, wsem_b)
  wait_writes(ng - 2, buf_a, wsem_a)
  wait_writes(ng - 1, buf_b, wsem_b)


def kernel(x, weight):
  n_rows, row_len = x.shape
  d = weight.shape[1]
  rows_per_w = n_rows // NUM_WORKERS

  mesh = plsc.VectorSubcoreMesh(core_axis_name="c", subcore_axis_name="s")
  run = pl.kernel(
      _gather_body,
      out_type=jax.ShapeDtypeStruct((n_rows, row_len, d), jnp.float32),
      mesh=mesh,
      scratch_types=[
          pltpu.VMEM((rows_per_w, row_len), jnp.int32),
          pltpu.VMEM((K, row_len, d), jnp.float32),
          pltpu.VMEM((K, row_len, d), jnp.float32),
          pltpu.SemaphoreType.DMA,
          pltpu.SemaphoreType.DMA,
          pltpu.SemaphoreType.DMA,
          pltpu.SemaphoreType.DMA,
      ],
      compiler_params=pltpu.CompilerParams(use_tc_tiling_on_sc=False),
  )
  return run(x, weight)


# speedup vs baseline: 1.1019x; 1.1019x over previous
import jax
import jax.numpy as jnp
from jax import lax
from jax.experimental import pallas as pl
from jax.experimental.pallas import tpu as pltpu
from jax.experimental.pallas import tpu_sc as plsc

NUM_WORKERS = 32
CHUNK = 128
K = 5
D = 64


def _gather_body(x_hbm, w_hbm, out_hbm, idx_v, buf_a, buf_b, gsem_a, gsem_b,
                 wsem_a, wsem_b):
  n_idx = x_hbm.shape[0] // NUM_WORKERS
  n_chunks = n_idx // CHUNK
  ng = n_chunks // K
  c = lax.axis_index("c")
  s = lax.axis_index("s")
  wid = s * 2 + c
  base = wid * n_idx
  pltpu.sync_copy(x_hbm.at[pl.ds(base, n_idx)], idx_v)

  def start_gathers(g, buf, sem):
    for b in range(K):
      pltpu.async_copy(
          w_hbm.at[idx_v.at[pl.ds((g * K + b) * CHUNK, CHUNK)]], buf.at[b],
          sem)

  def wait_gathers(g, buf, sem):
    for b in range(K):
      pltpu.make_async_copy(
          w_hbm.at[idx_v.at[pl.ds((g * K + b) * CHUNK, CHUNK)]], buf.at[b],
          sem).wait()

  def start_writes(g, buf, sem):
    for b in range(K):
      pltpu.async_copy(
          buf.at[b], out_hbm.at[pl.ds(base + (g * K + b) * CHUNK, CHUNK)],
          sem)

  def wait_writes(g, buf, sem):
    for b in range(K):
      pltpu.make_async_copy(
          buf.at[b], out_hbm.at[pl.ds(base + (g * K + b) * CHUNK, CHUNK)],
          sem).wait()

  start_gathers(0, buf_a, gsem_a)
  wait_gathers(0, buf_a, gsem_a)
  start_writes(0, buf_a, wsem_a)
  start_gathers(1, buf_b, gsem_b)
  wait_gathers(1, buf_b, gsem_b)
  start_writes(1, buf_b, wsem_b)
  wait_writes(0, buf_a, wsem_a)
  start_gathers(2, buf_a, gsem_a)

  def pair(q, carry):
    g = 2 * q + 2
    wait_gathers(g, buf_a, gsem_a)
    start_writes(g, buf_a, wsem_a)
    wait_writes(g - 1, buf_b, wsem_b)
    start_gathers(g + 1, buf_b, gsem_b)
    wait_gathers(g + 1, buf_b, gsem_b)
    start_writes(g + 1, buf_b, wsem_b)
    wait_writes(g, buf_a, wsem_a)
    start_gathers(g + 2, buf_a, gsem_a)
    return carry

  lax.fori_loop(0, (ng - 4) // 2, pair, 0)

  wait_gathers(ng - 2, buf_a, gsem_a)
  start_writes(ng - 2, buf_a, wsem_a)
  wait_writes(ng - 3, buf_b, wsem_b)
  start_gathers(ng - 1, buf_b, gsem_b)
  wait_gathers(ng - 1, buf_b, gsem_b)
  start_writes(ng - 1, buf_b, wsem_b)
  wait_writes(ng - 2, buf_a, wsem_a)
  wait_writes(ng - 1, buf_b, wsem_b)


def kernel(x, weight):
  n_emb = weight.shape[0]
  n_rows, row_len = x.shape
  n_idx = n_rows * row_len

  wp = jnp.pad(weight, ((0, 0), (0, 128 - D)))
  xf = (x.astype(jnp.int32) * 2).reshape(n_idx)

  mesh = plsc.VectorSubcoreMesh(core_axis_name="c", subcore_axis_name="s")
  run = pl.kernel(
      _gather_body,
      out_type=jax.ShapeDtypeStruct((n_idx, D), jnp.float32),
      mesh=mesh,
      scratch_types=[
          pltpu.VMEM((n_idx // NUM_WORKERS,), jnp.int32),
          pltpu.VMEM((K, CHUNK, D), jnp.float32),
          pltpu.VMEM((K, CHUNK, D), jnp.float32),
          pltpu.SemaphoreType.DMA,
          pltpu.SemaphoreType.DMA,
          pltpu.SemaphoreType.DMA,
          pltpu.SemaphoreType.DMA,
      ],
      compiler_params=pltpu.CompilerParams(use_tc_tiling_on_sc=False),
  )
  out = run(xf, wp.reshape(2 * n_emb, D))
  return out.reshape(n_rows, row_len, D)
